# Initial kernel scaffold; baseline (speedup 1.0000x reference)
import jax
import jax.numpy as jnp
from jax.experimental import pallas as pl


def kernel(x, edge_index, state, W1, b1, W2, b2, Ws, bs, Wo, bo):
    raise NotImplementedError("write your pallas kernel here")


# speedup vs baseline: 110.1766x; 110.1766x over previous
import functools

import jax
import jax.numpy as jnp
from jax import lax
from jax.experimental import pallas as pl
from jax.experimental.pallas import tpu as pltpu
from jax.experimental.pallas import tpu_sc as plsc

NC = 2
NS = 16
NW = NC * NS


def _pick_block(ept: int, maxb: int) -> int:
    for b in range(min(maxb, ept), 15, -1):
        if ept % b == 0 and b % 16 == 0:
            return b
    return 0


def _make_deg(n: int, e: int):
    ept = e // NW
    b1 = _pick_block(ept, 10000)
    e2 = e // NC
    mesh = plsc.VectorSubcoreMesh(core_axis_name="c", subcore_axis_name="s")

    @functools.partial(
        pl.kernel,
        out_type=jax.ShapeDtypeStruct((NC, n), jnp.float32),
        mesh=mesh,
        scratch_types=[
            pltpu.VMEM((b1,), jnp.int32),
            pltpu.VMEM((b1,), jnp.float32),
            pltpu.VMEM_SHARED((n,), jnp.float32),
        ],
        compiler_params=pltpu.CompilerParams(use_tc_tiling_on_sc=False),
    )
    def deg_kernel(dst_hbm, z1_hbm, outp, idx_v, ones_v, deg_sh):
        c = lax.axis_index("c")
        s = lax.axis_index("s")

        def fill(i, carry):
            ones_v[pl.ds(i * 16, 16)] = jnp.full((16,), 1.0, jnp.float32)
            return carry

        lax.fori_loop(0, b1 // 16, fill, 0)

        @pl.when(s == 0)
        def _():
            pltpu.sync_copy(z1_hbm, deg_sh)

        plsc.subcore_barrier()

        def step(i, carry):
            base = c * e2 + s * ept + i * b1
            pltpu.sync_copy(dst_hbm.at[pl.ds(base, b1)], idx_v)
            pltpu.sync_copy(ones_v, deg_sh.at[idx_v], add=True)
            return carry

        lax.fori_loop(0, ept // b1, step, 0)
        plsc.subcore_barrier()

        @pl.when(s == 0)
        def _():
            pltpu.sync_copy(deg_sh, outp.at[c])

    return deg_kernel


def _make_agg(n: int, e: int, f: int):
    ept = e // NW
    b3 = _pick_block(ept, 2000)
    e2 = e // NC
    mesh = plsc.VectorSubcoreMesh(core_axis_name="c", subcore_axis_name="s")

    @functools.partial(
        pl.kernel,
        out_type=(
            jax.ShapeDtypeStruct((NC, n, f), jnp.float32),
            jax.ShapeDtypeStruct((NC, n), jnp.float32),
        ),
        mesh=mesh,
        scratch_types=[
            pltpu.VMEM((b3,), jnp.int32),
            pltpu.VMEM((b3,), jnp.int32),
            pltpu.VMEM((b3, f), jnp.float32),
            pltpu.VMEM((b3,), jnp.float32),
            pltpu.VMEM_SHARED((n, f), jnp.float32),
            pltpu.VMEM_SHARED((n,), jnp.float32),
            pltpu.VMEM_SHARED((n, f), jnp.float32),
            pltpu.VMEM_SHARED((n,), jnp.float32),
            pltpu.SemaphoreType.DMA,
        ],
        compiler_params=pltpu.CompilerParams(use_tc_tiling_on_sc=False),
    )
    def agg_kernel(src_hbm, dst_hbm, y_hbm, dinv_hbm, z8_hbm, z1_hbm,
                   aggp, tp, sidx_v, didx_v, rows_v, dval_v,
                   agg_sh, t_sh, y_sh, dinv_sh, sem):
        c = lax.axis_index("c")
        s = lax.axis_index("s")

        @pl.when(s == 0)
        def _():
            pltpu.sync_copy(z8_hbm, agg_sh)
            pltpu.sync_copy(z1_hbm, t_sh)
            pltpu.sync_copy(y_hbm, y_sh)
            pltpu.sync_copy(dinv_hbm, dinv_sh)

        plsc.subcore_barrier()

        def step(i, carry):
            base = c * e2 + s * ept + i * b3
            pltpu.sync_copy(src_hbm.at[pl.ds(base, b3)], sidx_v)
            pltpu.sync_copy(dst_hbm.at[pl.ds(base, b3)], didx_v)
            pltpu.async_copy(y_sh.at[sidx_v], rows_v, sem).wait()
            pltpu.sync_copy(rows_v, agg_sh.at[didx_v], add=True)
            pltpu.async_copy(dinv_sh.at[didx_v], dval_v, sem).wait()
            pltpu.sync_copy(dval_v, t_sh.at[sidx_v], add=True)
            return carry

        lax.fori_loop(0, ept // b3, step, 0)
        plsc.subcore_barrier()

        @pl.when(s == 0)
        def _():
            pltpu.sync_copy(agg_sh, aggp.at[c])
            pltpu.sync_copy(t_sh, tp.at[c])

    return agg_kernel


def _make_dense(n: int, f: int, blocks: int):
    bn = n // blocks

    def dense_body(agg0, agg1, y, dinv2, w2, st,
                   w1r, b1r, w2r, b2r, wsr, bsr, wor, bor, out, acc):
        i = pl.program_id(0)

        @pl.when(i == 0)
        def _():
            acc[...] = jnp.zeros_like(acc)

        a = (agg0[...] + agg1[...] + y[...]) * dinv2[...]
        h1 = jnp.maximum(
            jnp.dot(a, w1r[...], preferred_element_type=jnp.float32)
            + b1r[...], 0.0)
        acc[...] += jnp.sum(h1 * w2[...], axis=0, keepdims=True)

        @pl.when(i == pl.num_programs(0) - 1)
        def _():
            g = jnp.dot(acc[...] * (1.0 / n), w2r[...],
                        preferred_element_type=jnp.float32) + b2r[...]
            se = jnp.maximum(
                jnp.dot(st[...], wsr[...],
                        preferred_element_type=jnp.float32) + bsr[...], 0.0)
            z = jnp.concatenate([g, se], axis=1)
            out[...] = jnp.dot(z, wor[...],
                               preferred_element_type=jnp.float32) + bor[...]

    full = lambda i: (0, 0)
    blk = lambda i: (i, 0)
    return pl.pallas_call(
        dense_body,
        grid=(blocks,),
        in_specs=[
            pl.BlockSpec((bn, f), blk),
            pl.BlockSpec((bn, f), blk),
            pl.BlockSpec((bn, f), blk),
            pl.BlockSpec((bn, 1), blk),
            pl.BlockSpec((bn, 1), blk),
            pl.BlockSpec((1, 8), full),
            pl.BlockSpec((f, 64), full),
            pl.BlockSpec((1, 64), full),
            pl.BlockSpec((64, 64), full),
            pl.BlockSpec((1, 64), full),
            pl.BlockSpec((8, 64), full),
            pl.BlockSpec((1, 64), full),
            pl.BlockSpec((128, 2), full),
            pl.BlockSpec((1, 2), full),
        ],
        out_specs=pl.BlockSpec((1, 2), full),
        out_shape=jax.ShapeDtypeStruct((1, 2), jnp.float32),
        scratch_shapes=[pltpu.VMEM((1, 64), jnp.float32)],
    )


def kernel(x, edge_index, state, W1, b1, W2, b2, Ws, bs, Wo, bo):
    n, f = x.shape
    e = edge_index.shape[1]
    src = edge_index[0]
    dst = edge_index[1]

    z1 = jnp.zeros((n,), jnp.float32)
    z8 = jnp.zeros((n, f), jnp.float32)

    degp = _make_deg(n, e)(dst, z1)
    deg = degp[0] + degp[1] + 1.0
    dinv = jax.lax.rsqrt(deg)
    y = x * dinv[:, None]

    aggp, tp = _make_agg(n, e, f)(src, dst, y, dinv, z8, z1)
    w = dinv * (tp[0] + tp[1] + dinv)

    blocks = 10 if n % 10 == 0 else 1
    out = _make_dense(n, f, blocks)(
        aggp[0], aggp[1], y, dinv[:, None], w[:, None], state,
        W1, b1[None], W2, b2[None], Ws, bs[None], Wo, bo[None])
    return out
